# baseline (speedup 1.0000x reference)
import jax
import jax.numpy as jnp
from jax import lax
from jax.experimental import pallas as pl
from jax.experimental.pallas import tpu as pltpu
from jax.experimental.pallas import tpu_sc as plsc

EMBED = 64
ENT = 1000000
B = 16384
NC, NS, L = 2, 16, 16
NW = NC * NS
ROWS = B // NW
CR = L
NCH = ROWS // CR
CL = CR * EMBED


def _body(idx_hbm, paths_hbm, ents_hbm,
          pos_out_hbm, neg_out_hbm,
          pidx_v, aidx_v, bidx_v,
          lp0, lp1, le0, le1, dp0, dp1, de0, de1,
          pos_s, neg_s, sem0, sem1):
    wid = lax.axis_index("s") * NC + lax.axis_index("c")
    base = wid * ROWS

    pltpu.sync_copy(idx_hbm.at[0, pl.ds(base, ROWS)], pidx_v)
    pltpu.sync_copy(idx_hbm.at[1, pl.ds(base, ROWS)], aidx_v)
    pltpu.sync_copy(idx_hbm.at[2, pl.ds(base, ROWS)], bidx_v)

    lp = (lp0, lp1)
    le = (le0, le1)
    dp = (dp0, dp1)
    de = (de0, de1)
    sems = (sem0, sem1)

    def build(ch, p):
        sl = pl.ds(ch * CR, L)
        rvp = pidx_v[sl]
        rva = aidx_v[sl]
        rvb = bidx_v[sl]
        lpb, leb = lp[p], le[p]

        def bld(d, carry):
            off = pl.multiple_of(d * L, L)
            dsc = d * ENT
            lpb[pl.ds(off, L)] = rvp + dsc
            leb[pl.ds(off, L)] = rva + dsc
            leb[pl.ds(CL + off, L)] = rvb + dsc
            return carry

        lax.fori_loop(0, EMBED, bld, 0)

    def fire(p):
        h1 = pltpu.async_copy(paths_hbm.at[lp[p]], dp[p], sems[p])
        h2 = pltpu.async_copy(ents_hbm.at[le[p]], de[p], sems[p])
        return h1, h2

    def compute(ch, p):
        dpb, deb = dp[p], de[p]

        def acc(d, carry):
            ap, an = carry
            off = pl.multiple_of(d * L, L)
            pv = dpb[pl.ds(off, L)]
            ap = ap + pv * deb[pl.ds(off, L)]
            an = an + pv * deb[pl.ds(CL + off, L)]
            return ap, an

        z = jnp.zeros((L,), jnp.float32)
        ap, an = lax.fori_loop(0, EMBED, acc, (z, z))
        out_sl = pl.ds(ch * CR, L)
        pos_s[out_sl] = ap
        neg_s[out_sl] = an

    build(0, 0)
    handles = [fire(0), None]
    for ch in range(NCH):
        p = ch % 2
        if ch + 1 < NCH:
            build(ch + 1, 1 - p)
            handles[1 - p] = fire(1 - p)
        h1, h2 = handles[p]
        h1.wait()
        h2.wait()
        compute(ch, p)

    pltpu.sync_copy(pos_s, pos_out_hbm.at[pl.ds(base, ROWS)])
    pltpu.sync_copy(neg_s, neg_out_hbm.at[pl.ds(base, ROWS)])


def kernel(ents_path_idxs, embeddings_entities, embeddings_paths):
    idx = ents_path_idxs.astype(jnp.int32).T
    ents_flat = embeddings_entities.T.reshape(EMBED * ENT)
    paths_flat = embeddings_paths.T.reshape(EMBED * ENT)

    mesh = plsc.VectorSubcoreMesh(core_axis_name="c", subcore_axis_name="s",
                                  num_cores=NC, num_subcores=NS)
    run = pl.kernel(
        _body,
        out_type=[jax.ShapeDtypeStruct((B,), jnp.float32),
                  jax.ShapeDtypeStruct((B,), jnp.float32)],
        mesh=mesh,
        compiler_params=pltpu.CompilerParams(use_tc_tiling_on_sc=False),
        scratch_types=[
            pltpu.VMEM((ROWS,), jnp.int32),
            pltpu.VMEM((ROWS,), jnp.int32),
            pltpu.VMEM((ROWS,), jnp.int32),
            pltpu.VMEM((CL,), jnp.int32),
            pltpu.VMEM((CL,), jnp.int32),
            pltpu.VMEM((2 * CL,), jnp.int32),
            pltpu.VMEM((2 * CL,), jnp.int32),
            pltpu.VMEM((CL,), jnp.float32),
            pltpu.VMEM((CL,), jnp.float32),
            pltpu.VMEM((2 * CL,), jnp.float32),
            pltpu.VMEM((2 * CL,), jnp.float32),
            pltpu.VMEM((ROWS,), jnp.float32),
            pltpu.VMEM((ROWS,), jnp.float32),
            pltpu.SemaphoreType.DMA,
            pltpu.SemaphoreType.DMA,
        ],
    )
    pos, neg = run(idx, paths_flat, ents_flat)
    return pos.reshape(B, 1), neg.reshape(B, 1)


# speedup vs baseline: 7.4460x; 7.4460x over previous
import jax
import jax.numpy as jnp
from jax import lax
from jax.experimental import pallas as pl
from jax.experimental.pallas import tpu as pltpu
from jax.experimental.pallas import tpu_sc as plsc

EMBED = 64
ENT = 1000000
B = 16384
NC, NS, L = 2, 16, 16
NW = NC * NS
ROWS = B // NW
TBS = 2048


def _tr_body(in_ref, out_ref):
    out_ref[...] = in_ref[...].T


def _transpose_table(tview):
    return pl.pallas_call(
        _tr_body,
        grid=(pl.cdiv(ENT, TBS),),
        in_specs=[pl.BlockSpec((EMBED, TBS), lambda j: (0, j))],
        out_specs=pl.BlockSpec((TBS, EMBED), lambda j: (j, 0)),
        out_shape=jax.ShapeDtypeStruct((ENT, EMBED), jnp.float32),
    )(tview)


def _body(idx_hbm, paths_hbm, ents_hbm,
          pos_out_hbm, neg_out_hbm,
          pidx_v, aidx_v, bidx_v, path_v, pos_v, neg_v,
          pos_s, neg_s, sem):
    wid = lax.axis_index("s") * NC + lax.axis_index("c")
    base = wid * ROWS

    pltpu.sync_copy(idx_hbm.at[0, pl.ds(base, ROWS)], pidx_v)
    pltpu.sync_copy(idx_hbm.at[1, pl.ds(base, ROWS)], aidx_v)
    pltpu.sync_copy(idx_hbm.at[2, pl.ds(base, ROWS)], bidx_v)

    h1 = pltpu.async_copy(paths_hbm.at[pidx_v], path_v, sem)
    h2 = pltpu.async_copy(ents_hbm.at[aidx_v], pos_v, sem)
    h3 = pltpu.async_copy(ents_hbm.at[bidx_v], neg_v, sem)
    h1.wait()
    h2.wait()
    h3.wait()

    iota = lax.iota(jnp.int32, L)
    perms = [iota ^ s for s in (8, 4, 2, 1)]

    def fold(v):
        for p in perms:
            v = v + v.at[p].get(mode="promise_in_bounds")
        return v

    def group(g, carry):
        posvec = jnp.zeros((L,), jnp.float32)
        negvec = jnp.zeros((L,), jnp.float32)
        for r in range(L):
            i = g * L + r
            ap = jnp.zeros((L,), jnp.float32)
            an = jnp.zeros((L,), jnp.float32)
            for k in range(EMBED // L):
                sl = pl.ds(k * L, L)
                pv = path_v[i, sl]
                ap = ap + pv * pos_v[i, sl]
                an = an + pv * neg_v[i, sl]
            lane = iota == r
            posvec = jnp.where(lane, fold(ap), posvec)
            negvec = jnp.where(lane, fold(an), negvec)
        out_sl = pl.ds(g * L, L)
        pos_s[out_sl] = posvec
        neg_s[out_sl] = negvec
        return carry

    lax.fori_loop(0, ROWS // L, group, 0)

    pltpu.sync_copy(pos_s, pos_out_hbm.at[pl.ds(base, ROWS)])
    pltpu.sync_copy(neg_s, neg_out_hbm.at[pl.ds(base, ROWS)])


def kernel(ents_path_idxs, embeddings_entities, embeddings_paths):
    idx = ents_path_idxs.astype(jnp.int32).T
    paths_rm = _transpose_table(embeddings_paths.T)

    mesh = plsc.VectorSubcoreMesh(core_axis_name="c", subcore_axis_name="s",
                                  num_cores=NC, num_subcores=NS)
    run = pl.kernel(
        _body,
        out_type=[jax.ShapeDtypeStruct((B,), jnp.float32),
                  jax.ShapeDtypeStruct((B,), jnp.float32)],
        mesh=mesh,
        compiler_params=pltpu.CompilerParams(use_tc_tiling_on_sc=False),
        scratch_types=[
            pltpu.VMEM((ROWS,), jnp.int32),
            pltpu.VMEM((ROWS,), jnp.int32),
            pltpu.VMEM((ROWS,), jnp.int32),
            pltpu.VMEM((ROWS, EMBED), jnp.float32),
            pltpu.VMEM((ROWS, EMBED), jnp.float32),
            pltpu.VMEM((ROWS, EMBED), jnp.float32),
            pltpu.VMEM((ROWS,), jnp.float32),
            pltpu.VMEM((ROWS,), jnp.float32),
            pltpu.SemaphoreType.DMA,
        ],
    )
    pos, neg = run(idx, paths_rm, embeddings_entities)
    return pos.reshape(B, 1), neg.reshape(B, 1)


# speedup vs baseline: 8.2765x; 1.1115x over previous
import jax
import jax.numpy as jnp
from jax import lax
from jax.experimental import pallas as pl
from jax.experimental.pallas import tpu as pltpu
from jax.experimental.pallas import tpu_sc as plsc

EMBED = 64
ENT = 1000000
B = 16384
NC, NS, L = 2, 16, 16
NW = NC * NS
ROWS = B // NW
TBS = 8192


def _tr_body(in_ref, out_ref):
    out_ref[...] = in_ref[...].T


def _transpose_table(tview):
    return pl.pallas_call(
        _tr_body,
        grid=(pl.cdiv(ENT, TBS),),
        in_specs=[pl.BlockSpec((EMBED, TBS), lambda j: (0, j))],
        out_specs=pl.BlockSpec((TBS, EMBED), lambda j: (j, 0)),
        out_shape=jax.ShapeDtypeStruct((ENT, EMBED), jnp.float32),
    )(tview)


def _body(idx_hbm, paths_hbm, ents_hbm,
          pos_out_hbm, neg_out_hbm,
          pidx_v, aidx_v, bidx_v, path_v, pos_v, neg_v,
          pos_s, neg_s, sem):
    wid = lax.axis_index("s") * NC + lax.axis_index("c")
    base = wid * ROWS

    pltpu.sync_copy(idx_hbm.at[0, pl.ds(base, ROWS)], pidx_v)
    pltpu.sync_copy(idx_hbm.at[1, pl.ds(base, ROWS)], aidx_v)
    pltpu.sync_copy(idx_hbm.at[2, pl.ds(base, ROWS)], bidx_v)

    h1 = pltpu.async_copy(paths_hbm.at[pidx_v], path_v, sem)
    h2 = pltpu.async_copy(ents_hbm.at[aidx_v], pos_v, sem)
    h3 = pltpu.async_copy(ents_hbm.at[bidx_v], neg_v, sem)
    h1.wait()
    h2.wait()
    h3.wait()

    iota = lax.iota(jnp.int32, L)
    perms = [iota ^ s for s in (8, 4, 2, 1)]

    def fold(v):
        for p in perms:
            v = v + v.at[p].get(mode="promise_in_bounds")
        return v

    def group(g, carry):
        posvec = jnp.zeros((L,), jnp.float32)
        negvec = jnp.zeros((L,), jnp.float32)
        for r in range(L):
            i = g * L + r
            ap = jnp.zeros((L,), jnp.float32)
            an = jnp.zeros((L,), jnp.float32)
            for k in range(EMBED // L):
                sl = pl.ds(k * L, L)
                pv = path_v[i, sl]
                ap = ap + pv * pos_v[i, sl]
                an = an + pv * neg_v[i, sl]
            lane = iota == r
            posvec = jnp.where(lane, fold(ap), posvec)
            negvec = jnp.where(lane, fold(an), negvec)
        out_sl = pl.ds(g * L, L)
        pos_s[out_sl] = posvec
        neg_s[out_sl] = negvec
        return carry

    lax.fori_loop(0, ROWS // L, group, 0)

    pltpu.sync_copy(pos_s, pos_out_hbm.at[pl.ds(base, ROWS)])
    pltpu.sync_copy(neg_s, neg_out_hbm.at[pl.ds(base, ROWS)])


def kernel(ents_path_idxs, embeddings_entities, embeddings_paths):
    idx = ents_path_idxs.astype(jnp.int32).T
    paths_rm = _transpose_table(embeddings_paths.T)

    mesh = plsc.VectorSubcoreMesh(core_axis_name="c", subcore_axis_name="s",
                                  num_cores=NC, num_subcores=NS)
    run = pl.kernel(
        _body,
        out_type=[jax.ShapeDtypeStruct((B,), jnp.float32),
                  jax.ShapeDtypeStruct((B,), jnp.float32)],
        mesh=mesh,
        compiler_params=pltpu.CompilerParams(use_tc_tiling_on_sc=False),
        scratch_types=[
            pltpu.VMEM((ROWS,), jnp.int32),
            pltpu.VMEM((ROWS,), jnp.int32),
            pltpu.VMEM((ROWS,), jnp.int32),
            pltpu.VMEM((ROWS, EMBED), jnp.float32),
            pltpu.VMEM((ROWS, EMBED), jnp.float32),
            pltpu.VMEM((ROWS, EMBED), jnp.float32),
            pltpu.VMEM((ROWS,), jnp.float32),
            pltpu.VMEM((ROWS,), jnp.float32),
            pltpu.SemaphoreType.DMA,
        ],
    )
    pos, neg = run(idx, paths_rm, embeddings_entities)
    return pos.reshape(B, 1), neg.reshape(B, 1)


# speedup vs baseline: 9.1077x; 1.1004x over previous
import jax
import jax.numpy as jnp
from jax import lax
from jax.experimental import pallas as pl
from jax.experimental.pallas import tpu as pltpu
from jax.experimental.pallas import tpu_sc as plsc

EMBED = 64
ENT = 1000000
B = 16384
NC, NS, L = 2, 16, 16
NW = NC * NS
ROWS = B // NW
CR = 128
NCH = ROWS // CR


def _body(idx_hbm, paths_hbm, ents_hbm,
          pos_out_hbm, neg_out_hbm,
          pidx_v, aidx_v, bidx_v, ppair_v, apair_v, bpair_v, parv,
          pb0, pb1, ab0, ab1, bb0, bb1, pos_s, neg_s, sem):
    wid = lax.axis_index("s") * NC + lax.axis_index("c")
    base = wid * ROWS

    pltpu.sync_copy(idx_hbm.at[pl.ds(base, ROWS)], pidx_v)
    pltpu.sync_copy(idx_hbm.at[pl.ds(B + base, ROWS)], aidx_v)
    pltpu.sync_copy(idx_hbm.at[pl.ds(2 * B + base, ROWS)], bidx_v)

    for t in range(ROWS // L):
        sl = pl.ds(t * L, L)
        pv = pidx_v[sl]
        av = aidx_v[sl]
        bv = bidx_v[sl]
        ppair_v[sl] = pv >> 1
        apair_v[sl] = av >> 1
        bpair_v[sl] = bv >> 1
        parv[sl] = (pv & 1) | ((av & 1) << 1) | ((bv & 1) << 2)

    pbufs, abufs, bbufs = (pb0, pb1), (ab0, ab1), (bb0, bb1)

    def fire(c):
        p = c & 1
        sl = pl.ds(c * CR, CR)
        return [pltpu.async_copy(paths_hbm.at[ppair_v.at[sl]], pbufs[p], sem),
                pltpu.async_copy(ents_hbm.at[apair_v.at[sl]], abufs[p], sem),
                pltpu.async_copy(ents_hbm.at[bpair_v.at[sl]], bbufs[p], sem)]

    iota = lax.iota(jnp.int32, L)
    perms = [iota ^ s for s in (8, 4, 2, 1)]

    def fold(v):
        for p in perms:
            v = v + v.at[p].get(mode="promise_in_bounds")
        return v

    def compute(c):
        p = c & 1
        pb, ab, bb = pbufs[p], abufs[p], bbufs[p]

        def group(g, carry):
            posvec = jnp.zeros((L,), jnp.float32)
            negvec = jnp.zeros((L,), jnp.float32)
            pvec = parv[pl.ds(c * CR + g * L, L)]
            for rr in range(L):
                i = g * L + rr
                bits = pvec.at[jnp.full((L,), rr, jnp.int32)].get(
                    mode="promise_in_bounds")
                mp = (bits & 1).astype(jnp.float32)
                ma = ((bits >> 1) & 1).astype(jnp.float32)
                mb = ((bits >> 2) & 1).astype(jnp.float32)
                ap = jnp.zeros((L,), jnp.float32)
                an = jnp.zeros((L,), jnp.float32)
                for k in range(EMBED // L):
                    lo = pl.ds(k * L, L)
                    hi = pl.ds(EMBED + k * L, L)
                    plo = pb[i, lo]
                    alo = ab[i, lo]
                    blo = bb[i, lo]
                    pvv = plo + mp * (pb[i, hi] - plo)
                    av = alo + ma * (ab[i, hi] - alo)
                    bv = blo + mb * (bb[i, hi] - blo)
                    ap = ap + pvv * av
                    an = an + pvv * bv
                lane = iota == rr
                posvec = jnp.where(lane, fold(ap), posvec)
                negvec = jnp.where(lane, fold(an), negvec)
            out_sl = pl.ds(c * CR + g * L, L)
            pos_s[out_sl] = posvec
            neg_s[out_sl] = negvec
            return carry

        lax.fori_loop(0, CR // L, group, 0)

    handles = fire(0)
    for c in range(NCH):
        if c + 1 < NCH:
            nxt = fire(c + 1)
        for h in handles:
            h.wait()
        compute(c)
        if c + 1 < NCH:
            handles = nxt

    pltpu.sync_copy(pos_s, pos_out_hbm.at[pl.ds(base, ROWS)])
    pltpu.sync_copy(neg_s, neg_out_hbm.at[pl.ds(base, ROWS)])


def kernel(ents_path_idxs, embeddings_entities, embeddings_paths):
    idx = ents_path_idxs.astype(jnp.int32).T.reshape(3 * B)
    ents2 = embeddings_entities.reshape(ENT // 2, 2 * EMBED)
    paths2 = embeddings_paths.reshape(ENT // 2, 2 * EMBED)

    mesh = plsc.VectorSubcoreMesh(core_axis_name="c", subcore_axis_name="s",
                                  num_cores=NC, num_subcores=NS)
    run = pl.kernel(
        _body,
        out_type=[jax.ShapeDtypeStruct((B,), jnp.float32),
                  jax.ShapeDtypeStruct((B,), jnp.float32)],
        mesh=mesh,
        compiler_params=pltpu.CompilerParams(use_tc_tiling_on_sc=True),
        scratch_types=[
            pltpu.VMEM((ROWS,), jnp.int32),
            pltpu.VMEM((ROWS,), jnp.int32),
            pltpu.VMEM((ROWS,), jnp.int32),
            pltpu.VMEM((ROWS,), jnp.int32),
            pltpu.VMEM((ROWS,), jnp.int32),
            pltpu.VMEM((ROWS,), jnp.int32),
            pltpu.VMEM((ROWS,), jnp.int32),
            pltpu.VMEM((CR, 2 * EMBED), jnp.float32),
            pltpu.VMEM((CR, 2 * EMBED), jnp.float32),
            pltpu.VMEM((CR, 2 * EMBED), jnp.float32),
            pltpu.VMEM((CR, 2 * EMBED), jnp.float32),
            pltpu.VMEM((CR, 2 * EMBED), jnp.float32),
            pltpu.VMEM((CR, 2 * EMBED), jnp.float32),
            pltpu.VMEM((ROWS,), jnp.float32),
            pltpu.VMEM((ROWS,), jnp.float32),
            pltpu.SemaphoreType.DMA,
        ],
    )
    pos, neg = run(idx, paths2, ents2)
    return pos.reshape(B, 1), neg.reshape(B, 1)
